# baseline (speedup 1.0000x reference)
import functools
import math

import jax
import jax.numpy as jnp
import numpy as np
from jax.experimental import pallas as pl

N = 4096
D = 128
OUT = 128

_K1 = (0x1E3F1835, 0x6E752082)
_K2 = (0x74298876, 0xFC8D8048)

_SQRTD = np.float32(np.sqrt(np.float32(D)))
_MINV = np.float32(1e-6)
_SPAN = np.float32(np.float32(1.0 - 1e-6) - np.float32(1e-6))
_ROTS = ((13, 15, 26, 6), (17, 29, 16, 24))

BM = 256
BR = 128
BC = 512
BR3 = 256
BC3 = 512


def _tf_gumbel(k0, k1, m):
    ks = (k0, k1, (k0 ^ k1 ^ 0x1BD11BDA) & 0xFFFFFFFF)
    x0 = jnp.full(m.shape, jnp.uint32(k0), jnp.uint32)
    x1 = m + jnp.uint32(k1)
    for g in range(5):
        for d in _ROTS[g % 2]:
            x0 = x0 + x1
            x1 = ((x1 << jnp.uint32(d)) | (x1 >> jnp.uint32(32 - d))) ^ x0
        x0 = x0 + jnp.uint32(ks[(g + 1) % 3])
        x1 = x1 + jnp.uint32((ks[(g + 2) % 3] + g + 1) & 0xFFFFFFFF)
    bits = x0 ^ x1
    fb = (bits >> jnp.uint32(9)) | jnp.uint32(0x3F800000)
    f = jax.lax.bitcast_convert_type(fb, jnp.float32)
    u = (f - jnp.float32(1.0)) * _SPAN + _MINV
    return -jnp.log(-jnp.log(u))


def _mlp_kernel(x_ref, w1_ref, b1_ref, w2_ref, b2_ref, wg_ref, h_ref, z_ref):
    h1 = jnp.maximum(jnp.dot(x_ref[...], w1_ref[...]) + b1_ref[...], 0.0)
    h = jnp.dot(h1, w2_ref[...]) + b2_ref[...]
    h_ref[...] = h
    z_ref[...] = jnp.dot(h, wg_ref[...])


def _adj_kernel(hr_ref, hc_ref, a_ref, deg_ref):
    i = pl.program_id(0)
    j = pl.program_id(1)
    p = jax.lax.dot_general(
        hr_ref[...], hc_ref[...], (((1,), (1,)), ((), ())),
        preferred_element_type=jnp.float32) / _SQRTD
    lin = (jax.lax.broadcasted_iota(jnp.int32, (BR, BC), 0) * N
           + jax.lax.broadcasted_iota(jnp.int32, (BR, BC), 1))
    m = lin.astype(jnp.uint32) + (i * (BR * N) + j * BC).astype(jnp.uint32)
    g1 = _tf_gumbel(_K1[0], _K1[1], m)
    g2 = _tf_gumbel(_K2[0], _K2[1], m)
    logits = (p + g1) - g2
    a = (logits > 0).astype(jnp.float32)

    def _emit(av):
        a_ref[...] = av.astype(jnp.bfloat16)
        rs = jnp.sum(av, axis=1, keepdims=True)

        @pl.when(j == 0)
        def _():
            deg_ref[...] = rs

        @pl.when(j != 0)
        def _():
            deg_ref[...] += rs

    on_diag = (i * BR >= j * BC) & (i * BR < (j + 1) * BC)

    @pl.when(on_diag)
    def _():
        rows = jax.lax.broadcasted_iota(jnp.int32, (BR, BC), 0) + i * BR
        cols = jax.lax.broadcasted_iota(jnp.int32, (BR, BC), 1) + j * BC
        _emit(jnp.where(rows == cols, jnp.float32(1.0), a))

    @pl.when(jnp.logical_not(on_diag))
    def _():
        _emit(a)


def _agg_kernel(a_ref, z_ref, degr_ref, degc_ref, bg_ref, out_ref):
    j = pl.program_id(1)
    nj = pl.num_programs(1)
    deg_c = degc_ref[...]
    dinv_c = jnp.where(deg_c > 0, jnp.float32(1.0) / jnp.sqrt(deg_c), 0.0)
    zd = (z_ref[...] * dinv_c).astype(jnp.bfloat16)
    contrib = jnp.dot(a_ref[...], zd, preferred_element_type=jnp.float32)

    @pl.when(j == 0)
    def _():
        out_ref[...] = contrib

    @pl.when(j != 0)
    def _():
        out_ref[...] += contrib

    @pl.when(j == nj - 1)
    def _():
        deg_r = degr_ref[...]
        dinv_r = jnp.where(deg_r > 0, jnp.float32(1.0) / jnp.sqrt(deg_r), 0.0)
        out_ref[...] = out_ref[...] * dinv_r + bg_ref[...]


@jax.jit
def kernel(x, W1, b1, W2, b2, Wg, bg):
    b1r = b1.reshape(1, D)
    b2r = b2.reshape(1, D)
    bgr = bg.reshape(1, OUT)

    h, z = pl.pallas_call(
        _mlp_kernel,
        grid=(N // BM,),
        in_specs=[
            pl.BlockSpec((BM, D), lambda i: (i, 0)),
            pl.BlockSpec((D, D), lambda i: (0, 0)),
            pl.BlockSpec((1, D), lambda i: (0, 0)),
            pl.BlockSpec((D, D), lambda i: (0, 0)),
            pl.BlockSpec((1, D), lambda i: (0, 0)),
            pl.BlockSpec((D, OUT), lambda i: (0, 0)),
        ],
        out_specs=[
            pl.BlockSpec((BM, D), lambda i: (i, 0)),
            pl.BlockSpec((BM, OUT), lambda i: (i, 0)),
        ],
        out_shape=[
            jax.ShapeDtypeStruct((N, D), jnp.float32),
            jax.ShapeDtypeStruct((N, OUT), jnp.float32),
        ],
    )(x, W1, b1r, W2, b2r, Wg)

    adj, deg = pl.pallas_call(
        _adj_kernel,
        grid=(N // BR, N // BC),
        in_specs=[
            pl.BlockSpec((BR, D), lambda i, j: (i, 0)),
            pl.BlockSpec((BC, D), lambda i, j: (j, 0)),
        ],
        out_specs=[
            pl.BlockSpec((BR, BC), lambda i, j: (i, j)),
            pl.BlockSpec((BR, 1), lambda i, j: (i, 0)),
        ],
        out_shape=[
            jax.ShapeDtypeStruct((N, N), jnp.bfloat16),
            jax.ShapeDtypeStruct((N, 1), jnp.float32),
        ],
    )(h, h)

    out = pl.pallas_call(
        _agg_kernel,
        grid=(N // BR3, N // BC3),
        in_specs=[
            pl.BlockSpec((BR3, BC3), lambda i, j: (i, j)),
            pl.BlockSpec((BC3, OUT), lambda i, j: (j, 0)),
            pl.BlockSpec((BR3, 1), lambda i, j: (i, 0)),
            pl.BlockSpec((BC3, 1), lambda i, j: (j, 0)),
            pl.BlockSpec((1, OUT), lambda i, j: (0, 0)),
        ],
        out_specs=pl.BlockSpec((BR3, OUT), lambda i, j: (i, 0)),
        out_shape=jax.ShapeDtypeStruct((N, OUT), jnp.float32),
    )(adj, z, deg, deg, bgr)

    return out


# speedup vs baseline: 1.4438x; 1.4438x over previous
import functools
import math

import jax
import jax.numpy as jnp
import numpy as np
from jax.experimental import pallas as pl

N = 4096
D = 128
OUT = 128

_K1 = (0x1E3F1835, 0x6E752082)
_K2 = (0x74298876, 0xFC8D8048)

_SQRTD = np.float32(np.sqrt(np.float32(D)))
_MINV = np.float32(1e-6)
_SPAN = np.float32(np.float32(1.0 - 1e-6) - np.float32(1e-6))
_ROTS = ((13, 15, 26, 6), (17, 29, 16, 24))

BM = 256
BR = 256
BC = 1024
BR3 = 256
BC3 = 512


def _tf_gumbel(k0, k1, m):
    ks = (k0, k1, (k0 ^ k1 ^ 0x1BD11BDA) & 0xFFFFFFFF)
    x0 = jnp.full(m.shape, jnp.uint32(k0), jnp.uint32)
    x1 = m + jnp.uint32(k1)
    for g in range(5):
        for d in _ROTS[g % 2]:
            x0 = x0 + x1
            x1 = (x1 * jnp.uint32(1 << d) + (x1 >> jnp.uint32(32 - d))) ^ x0
        x0 = x0 + jnp.uint32(ks[(g + 1) % 3])
        x1 = x1 + jnp.uint32((ks[(g + 2) % 3] + g + 1) & 0xFFFFFFFF)
    bits = x0 ^ x1
    fb = (bits >> jnp.uint32(9)) | jnp.uint32(0x3F800000)
    f = jax.lax.bitcast_convert_type(fb, jnp.float32)
    u = (f - jnp.float32(1.0)) * _SPAN + _MINV
    return -jnp.log(-jnp.log(u))


def _mlp_kernel(x_ref, w1_ref, b1_ref, w2_ref, b2_ref, wg_ref, h_ref, z_ref):
    h1 = jnp.maximum(jnp.dot(x_ref[...], w1_ref[...]) + b1_ref[...], 0.0)
    h = jnp.dot(h1, w2_ref[...]) + b2_ref[...]
    h_ref[...] = h
    z_ref[...] = jnp.dot(h, wg_ref[...])


def _adj_kernel(hr_ref, hc_ref, a_ref, deg_ref):
    i = pl.program_id(0)
    j = pl.program_id(1)
    p = jax.lax.dot_general(
        hr_ref[...], hc_ref[...], (((1,), (1,)), ((), ())),
        preferred_element_type=jnp.float32) / _SQRTD
    lin = (jax.lax.broadcasted_iota(jnp.int32, (BR, BC), 0) * N
           + jax.lax.broadcasted_iota(jnp.int32, (BR, BC), 1))
    m = lin.astype(jnp.uint32) + (i * (BR * N) + j * BC).astype(jnp.uint32)
    g1 = _tf_gumbel(_K1[0], _K1[1], m)
    g2 = _tf_gumbel(_K2[0], _K2[1], m)
    logits = (p + g1) - g2
    on_diag = (m >> jnp.uint32(12)) == (m & jnp.uint32(N - 1))
    a = jnp.where(on_diag, jnp.float32(1.0),
                  (logits > 0).astype(jnp.float32))
    a_ref[...] = a.astype(jnp.bfloat16)
    rs = jnp.sum(a, axis=1, keepdims=True)

    @pl.when(j == 0)
    def _():
        deg_ref[...] = rs

    @pl.when(j != 0)
    def _():
        deg_ref[...] += rs


def _prep_kernel(deg_ref, z_ref, dinv_ref, zd_ref):
    deg = deg_ref[...]
    dinv = jnp.where(deg > 0, jnp.float32(1.0) / jnp.sqrt(deg), 0.0)
    dinv_ref[...] = dinv
    zd_ref[...] = (z_ref[...] * dinv).astype(jnp.bfloat16)


def _agg_kernel(a_ref, zd_ref, dinvr_ref, bg_ref, out_ref):
    j = pl.program_id(1)
    nj = pl.num_programs(1)
    contrib = jnp.dot(a_ref[...], zd_ref[...], preferred_element_type=jnp.float32)

    @pl.when(j == 0)
    def _():
        out_ref[...] = contrib

    @pl.when(j != 0)
    def _():
        out_ref[...] += contrib

    @pl.when(j == nj - 1)
    def _():
        out_ref[...] = out_ref[...] * dinvr_ref[...] + bg_ref[...]


@jax.jit
def kernel(x, W1, b1, W2, b2, Wg, bg):
    b1r = b1.reshape(1, D)
    b2r = b2.reshape(1, D)
    bgr = bg.reshape(1, OUT)

    h, z = pl.pallas_call(
        _mlp_kernel,
        grid=(N // BM,),
        in_specs=[
            pl.BlockSpec((BM, D), lambda i: (i, 0)),
            pl.BlockSpec((D, D), lambda i: (0, 0)),
            pl.BlockSpec((1, D), lambda i: (0, 0)),
            pl.BlockSpec((D, D), lambda i: (0, 0)),
            pl.BlockSpec((1, D), lambda i: (0, 0)),
            pl.BlockSpec((D, OUT), lambda i: (0, 0)),
        ],
        out_specs=[
            pl.BlockSpec((BM, D), lambda i: (i, 0)),
            pl.BlockSpec((BM, OUT), lambda i: (i, 0)),
        ],
        out_shape=[
            jax.ShapeDtypeStruct((N, D), jnp.float32),
            jax.ShapeDtypeStruct((N, OUT), jnp.float32),
        ],
    )(x, W1, b1r, W2, b2r, Wg)

    adj, deg = pl.pallas_call(
        _adj_kernel,
        grid=(N // BR, N // BC),
        in_specs=[
            pl.BlockSpec((BR, D), lambda i, j: (i, 0)),
            pl.BlockSpec((BC, D), lambda i, j: (j, 0)),
        ],
        out_specs=[
            pl.BlockSpec((BR, BC), lambda i, j: (i, j)),
            pl.BlockSpec((BR, 1), lambda i, j: (i, 0)),
        ],
        out_shape=[
            jax.ShapeDtypeStruct((N, N), jnp.bfloat16),
            jax.ShapeDtypeStruct((N, 1), jnp.float32),
        ],
    )(h, h)

    dinv, zd = pl.pallas_call(
        _prep_kernel,
        grid=(N // BM,),
        in_specs=[
            pl.BlockSpec((BM, 1), lambda i: (i, 0)),
            pl.BlockSpec((BM, OUT), lambda i: (i, 0)),
        ],
        out_specs=[
            pl.BlockSpec((BM, 1), lambda i: (i, 0)),
            pl.BlockSpec((BM, OUT), lambda i: (i, 0)),
        ],
        out_shape=[
            jax.ShapeDtypeStruct((N, 1), jnp.float32),
            jax.ShapeDtypeStruct((N, OUT), jnp.bfloat16),
        ],
    )(deg, z)

    out = pl.pallas_call(
        _agg_kernel,
        grid=(N // BR3, N // BC3),
        in_specs=[
            pl.BlockSpec((BR3, BC3), lambda i, j: (i, j)),
            pl.BlockSpec((BC3, OUT), lambda i, j: (j, 0)),
            pl.BlockSpec((BR3, 1), lambda i, j: (i, 0)),
            pl.BlockSpec((1, OUT), lambda i, j: (0, 0)),
        ],
        out_specs=pl.BlockSpec((BR3, OUT), lambda i, j: (i, 0)),
        out_shape=jax.ShapeDtypeStruct((N, OUT), jnp.float32),
    )(adj, zd, dinv, bgr)

    return out


# speedup vs baseline: 1.6192x; 1.1215x over previous
import jax
import jax.numpy as jnp
import numpy as np
from jax.experimental import pallas as pl
from jax.experimental.pallas import tpu as pltpu

N = 4096
D = 128
OUT = 128

_K1 = (0x1E3F1835, 0x6E752082)
_K2 = (0x74298876, 0xFC8D8048)

_SQRTD = np.float32(np.sqrt(np.float32(D)))
_MINV = np.float32(1e-6)
_SPAN = np.float32(np.float32(1.0 - 1e-6) - np.float32(1e-6))
_ROTS = ((13, 15, 26, 6), (17, 29, 16, 24))

BR = 256
BC = 1024
BR3 = 256
BC3 = 512


def _tf_gumbel(k0, k1, m):
    ks = (k0, k1, (k0 ^ k1 ^ 0x1BD11BDA) & 0xFFFFFFFF)
    x0 = jnp.full(m.shape, jnp.uint32(k0), jnp.uint32)
    x1 = m + jnp.uint32(k1)
    for g in range(5):
        for d in _ROTS[g % 2]:
            x0 = x0 + x1
            x1 = (x1 * jnp.uint32(1 << d) + (x1 >> jnp.uint32(32 - d))) ^ x0
        x0 = x0 + jnp.uint32(ks[(g + 1) % 3])
        x1 = x1 + jnp.uint32((ks[(g + 2) % 3] + g + 1) & 0xFFFFFFFF)
    bits = x0 ^ x1
    fb = (bits >> jnp.uint32(9)) | jnp.uint32(0x3F800000)
    f = jax.lax.bitcast_convert_type(fb, jnp.float32)
    u = (f - jnp.float32(1.0)) * _SPAN + _MINV
    return -jnp.log(-jnp.log(u))


def _mlp(xv, w1_ref, b1_ref, w2_ref, b2_ref):
    h1 = jnp.maximum(jnp.dot(xv, w1_ref[...]) + b1_ref[...], 0.0)
    return jnp.dot(h1, w2_ref[...]) + b2_ref[...]


def _adj_kernel(xr_ref, xc_ref, w1_ref, b1_ref, w2_ref, b2_ref, wg_ref,
                a_ref, deg_ref, z_ref):
    i = pl.program_id(0)
    j = pl.program_id(1)
    h_r = _mlp(xr_ref[...], w1_ref, b1_ref, w2_ref, b2_ref)
    h_c = _mlp(xc_ref[...], w1_ref, b1_ref, w2_ref, b2_ref)
    z_ref[...] = jnp.dot(h_c, wg_ref[...])
    p = jax.lax.dot_general(
        h_r, h_c, (((1,), (1,)), ((), ())),
        preferred_element_type=jnp.float32) / _SQRTD
    lin = (jax.lax.broadcasted_iota(jnp.int32, (BR, BC), 0) * N
           + jax.lax.broadcasted_iota(jnp.int32, (BR, BC), 1))
    m = lin.astype(jnp.uint32) + (i * (BR * N) + j * BC).astype(jnp.uint32)
    g1 = _tf_gumbel(_K1[0], _K1[1], m)
    g2 = _tf_gumbel(_K2[0], _K2[1], m)
    logits = (p + g1) - g2
    on_diag = (m >> jnp.uint32(12)) == (m & jnp.uint32(N - 1))
    a = jnp.where(on_diag, jnp.float32(1.0),
                  (logits > 0).astype(jnp.float32))
    a_ref[...] = a.astype(jnp.bfloat16)
    rs = jnp.sum(a, axis=1, keepdims=True)

    @pl.when(j == 0)
    def _():
        deg_ref[...] = rs

    @pl.when(j != 0)
    def _():
        deg_ref[...] += rs


def _agg_kernel(a_ref, z_ref, deg_ref, bg_ref, out_ref, zd_ref, dinv_ref):
    i = pl.program_id(0)

    @pl.when(i == 0)
    def _():
        deg = deg_ref[...]
        dinv = jnp.where(deg > 0, jnp.float32(1.0) / jnp.sqrt(deg), 0.0)
        dinv_ref[...] = dinv
        zd_ref[...] = (z_ref[...] * dinv).astype(jnp.bfloat16)

    contrib = jnp.dot(a_ref[...], zd_ref[...],
                      preferred_element_type=jnp.float32)
    dinv_r = dinv_ref[pl.ds(i * BR3, BR3), :]
    out_ref[...] = contrib * dinv_r + bg_ref[...]


@jax.jit
def kernel(x, W1, b1, W2, b2, Wg, bg):
    b1r = b1.reshape(1, D)
    b2r = b2.reshape(1, D)
    bgr = bg.reshape(1, OUT)

    adj, deg, z = pl.pallas_call(
        _adj_kernel,
        grid=(N // BR, N // BC),
        in_specs=[
            pl.BlockSpec((BR, D), lambda i, j: (i, 0)),
            pl.BlockSpec((BC, D), lambda i, j: (j, 0)),
            pl.BlockSpec((D, D), lambda i, j: (0, 0)),
            pl.BlockSpec((1, D), lambda i, j: (0, 0)),
            pl.BlockSpec((D, D), lambda i, j: (0, 0)),
            pl.BlockSpec((1, D), lambda i, j: (0, 0)),
            pl.BlockSpec((D, OUT), lambda i, j: (0, 0)),
        ],
        out_specs=[
            pl.BlockSpec((BR, BC), lambda i, j: (i, j)),
            pl.BlockSpec((BR, 1), lambda i, j: (i, 0)),
            pl.BlockSpec((BC, OUT), lambda i, j: (j, 0)),
        ],
        out_shape=[
            jax.ShapeDtypeStruct((N, N), jnp.bfloat16),
            jax.ShapeDtypeStruct((N, 1), jnp.float32),
            jax.ShapeDtypeStruct((N, OUT), jnp.float32),
        ],
    )(x, x, W1, b1r, W2, b2r, Wg)

    out = pl.pallas_call(
        _agg_kernel,
        grid=(N // BR3,),
        in_specs=[
            pl.BlockSpec((BR3, N), lambda i: (i, 0)),
            pl.BlockSpec((N, OUT), lambda i: (0, 0)),
            pl.BlockSpec((N, 1), lambda i: (0, 0)),
            pl.BlockSpec((1, OUT), lambda i: (0, 0)),
        ],
        out_specs=pl.BlockSpec((BR3, OUT), lambda i: (i, 0)),
        out_shape=jax.ShapeDtypeStruct((N, OUT), jnp.float32),
        scratch_shapes=[
            pltpu.VMEM((N, OUT), jnp.bfloat16),
            pltpu.VMEM((N, 1), jnp.float32),
        ],
    )(adj, z, deg, bgr)

    return out
